# Initial kernel scaffold; baseline (speedup 1.0000x reference)
import jax
import jax.numpy as jnp
from jax.experimental import pallas as pl


def kernel(positions, type_indices, edge_index, emb_table, W1, b1, W2, b2, W3, b3):
    raise NotImplementedError("write your pallas kernel here")


# speedup vs baseline: 14.0774x; 14.0774x over previous
import functools

import jax
import jax.numpy as jnp
from jax import lax
from jax.experimental import pallas as pl
from jax.experimental.pallas import tpu as pltpu
from jax.experimental.pallas import tpu_sc as plsc

_CUTOFF = 0.09
_NW = 32
_LANES = 16
_BLK = 2048


def _sc_edge_gather(posx, posy, posz, typ, src, dst, n_elem):
    e_pad = src.shape[0]
    chunk = e_pad // _NW
    steps = chunk // _LANES
    n_atoms = posx.shape[0]

    def body(posx_h, posy_h, posz_h, typ_h, src_h, dst_h, r2_h, pr_h,
             px, py, pz, ty, sv, dv, r2v, prv):
        cid = lax.axis_index("c")
        sid = lax.axis_index("s")
        wid = cid * 16 + sid
        base = wid * chunk
        pltpu.sync_copy(posx_h, px)
        pltpu.sync_copy(posy_h, py)
        pltpu.sync_copy(posz_h, pz)
        pltpu.sync_copy(typ_h, ty)
        pltpu.sync_copy(src_h.at[pl.ds(base, chunk)], sv)
        pltpu.sync_copy(dst_h.at[pl.ds(base, chunk)], dv)

        def step(i, carry):
            sl = pl.ds(i * _LANES, _LANES)
            s = sv[sl]
            d = dv[sl]
            xs = plsc.load_gather(px, [s])
            xd = plsc.load_gather(px, [d])
            ys = plsc.load_gather(py, [s])
            yd = plsc.load_gather(py, [d])
            zs = plsc.load_gather(pz, [s])
            zd = plsc.load_gather(pz, [d])
            ts = plsc.load_gather(ty, [s])
            td = plsc.load_gather(ty, [d])
            ddx = xd - xs
            ddy = yd - ys
            ddz = zd - zs
            r2v[sl] = ddx * ddx + ddy * ddy + ddz * ddz
            prv[sl] = td * n_elem + ts
            return carry

        lax.fori_loop(0, steps, step, 0)
        pltpu.sync_copy(r2v, r2_h.at[pl.ds(base, chunk)])
        pltpu.sync_copy(prv, pr_h.at[pl.ds(base, chunk)])

    mesh = plsc.VectorSubcoreMesh(core_axis_name="c", subcore_axis_name="s")
    fn = pl.kernel(
        body,
        mesh=mesh,
        compiler_params=pltpu.CompilerParams(needs_layout_passes=False),
        out_type=(
            jax.ShapeDtypeStruct((e_pad,), jnp.float32),
            jax.ShapeDtypeStruct((e_pad,), jnp.int32),
        ),
        scratch_types=[
            pltpu.VMEM((n_atoms,), jnp.float32),
            pltpu.VMEM((n_atoms,), jnp.float32),
            pltpu.VMEM((n_atoms,), jnp.float32),
            pltpu.VMEM((n_atoms,), jnp.int32),
            pltpu.VMEM((chunk,), jnp.int32),
            pltpu.VMEM((chunk,), jnp.int32),
            pltpu.VMEM((chunk,), jnp.float32),
            pltpu.VMEM((chunk,), jnp.int32),
        ],
    )
    return fn(posx, posy, posz, typ, src, dst)


def _tc_energy(r2p, prp, wcat, w2, b2c, w3c, b3c, n_edges):
    e_pad = r2p.shape[0]
    grid = e_pad // _BLK
    hid = w2.shape[0]

    def body(r2_ref, pr_ref, wcat_ref, w2_ref, b2_ref, w3_ref, b3_ref, out_ref):
        pid = pl.program_id(0)

        @pl.when(pid == 0)
        def _():
            out_ref[0, 0] = 0.0

        r2 = r2_ref[...]
        pr = pr_ref[...]
        r = jnp.sqrt(r2)
        x = r * (1.0 / _CUTOFF)
        x2 = x * x
        x3 = x2 * x
        x6 = x3 * x3
        x7 = x6 * x
        x8 = x7 * x
        env = 1.0 - 28.0 * x6 + 48.0 * x7 - 21.0 * x8
        env = jnp.where(x < 1.0, env, 0.0)
        scale = jnp.sqrt(2.0 / _CUTOFF) * env / jnp.maximum(r, 1e-12)
        prow = lax.broadcasted_iota(jnp.int32, (16, _BLK), 0)
        nrow = (prow + 1).astype(jnp.float32)
        bess = jnp.sin(nrow * (jnp.pi * x)) * scale
        oneh = (prow == pr).astype(jnp.float32)
        feat = jnp.concatenate([bess, oneh], axis=0)
        pre1 = jnp.dot(wcat_ref[...], feat,
                       preferred_element_type=jnp.float32)
        h1 = pre1 * jax.nn.sigmoid(pre1)
        pre2 = jnp.dot(w2_ref[...], h1,
                       preferred_element_type=jnp.float32) + b2_ref[...]
        h2 = pre2 * jax.nn.sigmoid(pre2)
        evec = jnp.sum(w3_ref[...] * h2, axis=0, keepdims=True) + b3_ref[0, 0]
        eidx = pid * _BLK + lax.broadcasted_iota(jnp.int32, (1, _BLK), 1)
        e = jnp.where(eidx < n_edges, evec, 0.0)
        out_ref[0, 0] += 0.5 * jnp.sum(e)

    out = pl.pallas_call(
        body,
        grid=(grid,),
        in_specs=[
            pl.BlockSpec((1, _BLK), lambda i: (0, i)),
            pl.BlockSpec((1, _BLK), lambda i: (0, i)),
            pl.BlockSpec((hid, 2 * hid), lambda i: (0, 0)),
            pl.BlockSpec((hid, hid), lambda i: (0, 0)),
            pl.BlockSpec((hid, 1), lambda i: (0, 0)),
            pl.BlockSpec((hid, 1), lambda i: (0, 0)),
            pl.BlockSpec(memory_space=pltpu.SMEM),
        ],
        out_specs=pl.BlockSpec(memory_space=pltpu.SMEM),
        out_shape=jax.ShapeDtypeStruct((1, 1), jnp.float32),
    )(r2p.reshape(1, e_pad), prp.reshape(1, e_pad), wcat, w2, b2c, w3c, b3c)
    return out


def kernel(positions, type_indices, edge_index, emb_table, W1, b1, W2, b2, W3, b3):
    n_edges = edge_index.shape[1]
    e_pad = ((n_edges + _BLK - 1) // _BLK) * _BLK
    pad = e_pad - n_edges
    src = jnp.concatenate([edge_index[0], jnp.zeros((pad,), jnp.int32)])
    dst = jnp.concatenate([edge_index[1], jnp.zeros((pad,), jnp.int32)])
    posx = positions[:, 0]
    posy = positions[:, 1]
    posz = positions[:, 2]

    ne = emb_table.shape[0]
    td_dim = emb_table.shape[1]
    hid = W1.shape[0]
    a_d = emb_table @ W1[:, :td_dim].T
    b_s = emb_table @ W1[:, td_dim:2 * td_dim].T
    pt = (a_d[:, None, :] + b_s[None, :, :] + b1).reshape(ne * ne, hid)
    wr = jnp.pad(W1[:, 2 * td_dim:], ((0, 0), (0, 16 - (W1.shape[1] - 2 * td_dim))))
    wcat = jnp.concatenate([wr, pt.T], axis=1)

    r2p, prp = _sc_edge_gather(posx, posy, posz, type_indices, src, dst, ne)
    out = _tc_energy(r2p, prp, wcat, W2, b2.reshape(hid, 1),
                     W3.reshape(hid, 1), b3.reshape(1, 1), n_edges)
    return out[0, 0]


# speedup vs baseline: 32.3860x; 2.3006x over previous
import functools

import jax
import jax.numpy as jnp
from jax import lax
from jax.experimental import pallas as pl
from jax.experimental.pallas import tpu as pltpu
from jax.experimental.pallas import tpu_sc as plsc

_CUTOFF = 0.09
_NW = 32
_LANES = 16
_BLK = 2048


def _sc_edge_gather(posx, posy, posz, typ, src, dst, n_elem):
    e_pad = src.shape[0]
    chunk = e_pad // _NW
    steps = chunk // _LANES
    n_atoms = posx.shape[0]

    def body(posx_h, posy_h, posz_h, typ_h, src_h, dst_h, r2_h, pr_h,
             px, py, pz, ty, sv, dv, r2v, prv):
        cid = lax.axis_index("c")
        sid = lax.axis_index("s")
        wid = cid * 16 + sid
        base = wid * chunk
        pltpu.sync_copy(posx_h, px)
        pltpu.sync_copy(posy_h, py)
        pltpu.sync_copy(posz_h, pz)
        pltpu.sync_copy(typ_h, ty)
        pltpu.sync_copy(src_h.at[pl.ds(base, chunk)], sv)
        pltpu.sync_copy(dst_h.at[pl.ds(base, chunk)], dv)

        @plsc.parallel_loop(0, steps, step=1, unroll=8)
        def _step(i):
            sl = pl.ds(i * _LANES, _LANES)
            s = sv[sl]
            d = dv[sl]
            xs = plsc.load_gather(px, [s])
            xd = plsc.load_gather(px, [d])
            ys = plsc.load_gather(py, [s])
            yd = plsc.load_gather(py, [d])
            zs = plsc.load_gather(pz, [s])
            zd = plsc.load_gather(pz, [d])
            ts = plsc.load_gather(ty, [s])
            td = plsc.load_gather(ty, [d])
            ddx = xd - xs
            ddy = yd - ys
            ddz = zd - zs
            r2v[sl] = ddx * ddx + ddy * ddy + ddz * ddz
            prv[sl] = td * n_elem + ts
        pltpu.sync_copy(r2v, r2_h.at[pl.ds(base, chunk)])
        pltpu.sync_copy(prv, pr_h.at[pl.ds(base, chunk)])

    mesh = plsc.VectorSubcoreMesh(core_axis_name="c", subcore_axis_name="s")
    fn = pl.kernel(
        body,
        mesh=mesh,
        compiler_params=pltpu.CompilerParams(needs_layout_passes=False),
        out_type=(
            jax.ShapeDtypeStruct((e_pad,), jnp.float32),
            jax.ShapeDtypeStruct((e_pad,), jnp.int32),
        ),
        scratch_types=[
            pltpu.VMEM((n_atoms,), jnp.float32),
            pltpu.VMEM((n_atoms,), jnp.float32),
            pltpu.VMEM((n_atoms,), jnp.float32),
            pltpu.VMEM((n_atoms,), jnp.int32),
            pltpu.VMEM((chunk,), jnp.int32),
            pltpu.VMEM((chunk,), jnp.int32),
            pltpu.VMEM((chunk,), jnp.float32),
            pltpu.VMEM((chunk,), jnp.int32),
        ],
    )
    return fn(posx, posy, posz, typ, src, dst)


_CB = 4096
_BMAX = 10


def _tc_energy(r2p, prp, w1k, w2k, b2k, w3k, b3c, n_edges, ne):
    e_pad = r2p.shape[0]
    cols = e_pad // 8
    grid = cols // _CB
    nfeat = _BMAX + 2 * ne

    def body(r2_ref, pr_ref, w1_ref, w2_ref, b2_ref, w3_ref, b3_ref, out_ref):
        pid = pl.program_id(0)

        @pl.when(pid == 0)
        def _():
            out_ref[0, 0] = 0.0

        r2 = r2_ref[...]
        pr = pr_ref[...]
        r = jnp.sqrt(r2)
        x = r * (1.0 / _CUTOFF)
        x2 = x * x
        x3 = x2 * x
        x6 = x3 * x3
        x7 = x6 * x
        x8 = x7 * x
        env = 1.0 - 28.0 * x6 + 48.0 * x7 - 21.0 * x8
        env = jnp.where(x < 1.0, env, 0.0)
        scale = jnp.sqrt(2.0 / _CUTOFF) * env / jnp.maximum(r, 1e-12)
        ang = jnp.pi * x
        c1 = jnp.cos(ang)
        two_c1 = c1 + c1
        s = [jnp.sin(ang) * scale]
        sm1 = jnp.zeros_like(r2)
        for _n in range(_BMAX - 1):
            s_next = two_c1 * s[-1] - sm1
            sm1 = s[-1]
            s.append(s_next)
        td = pr // ne
        ts = pr - td * ne
        oh = [(td == t).astype(jnp.float32) for t in range(ne)]
        oh += [(ts == t).astype(jnp.float32) for t in range(ne)]
        feat = jnp.concatenate(s + oh, axis=0)
        pre1 = jnp.dot(w1_ref[...], feat,
                       preferred_element_type=jnp.float32)
        h1 = pre1 * jax.nn.sigmoid(pre1)
        pre2 = jnp.dot(w2_ref[...], h1,
                       preferred_element_type=jnp.float32) + b2_ref[...]
        h2 = pre2 * jax.nn.sigmoid(pre2)
        evec = jnp.dot(w3_ref[...], h2,
                       preferred_element_type=jnp.float32)
        jrow = lax.broadcasted_iota(jnp.int32, (8, _CB), 0) * cols
        lane = lax.broadcasted_iota(jnp.int32, (8, _CB), 1)
        eid = jrow + pid * _CB + lane
        e = jnp.where(eid < n_edges, evec + b3_ref[0, 0], 0.0)
        out_ref[0, 0] += 0.5 * jnp.sum(e)

    out = pl.pallas_call(
        body,
        grid=(grid,),
        in_specs=[
            pl.BlockSpec((8, _CB), lambda i: (0, i)),
            pl.BlockSpec((8, _CB), lambda i: (0, i)),
            pl.BlockSpec((128, 8 * nfeat), lambda i: (0, 0)),
            pl.BlockSpec((128, 128), lambda i: (0, 0)),
            pl.BlockSpec((128, 1), lambda i: (0, 0)),
            pl.BlockSpec((8, 128), lambda i: (0, 0)),
            pl.BlockSpec(memory_space=pltpu.SMEM),
        ],
        out_specs=pl.BlockSpec(memory_space=pltpu.SMEM),
        out_shape=jax.ShapeDtypeStruct((1, 1), jnp.float32),
    )(r2p.reshape(8, cols), prp.reshape(8, cols), w1k, w2k, b2k, w3k, b3c)
    return out


def kernel(positions, type_indices, edge_index, emb_table, W1, b1, W2, b2, W3, b3):
    n_edges = edge_index.shape[1]
    blk_edges = 8 * _CB
    e_pad = ((n_edges + blk_edges - 1) // blk_edges) * blk_edges
    pad = e_pad - n_edges
    src = jnp.concatenate([edge_index[0], jnp.zeros((pad,), jnp.int32)])
    dst = jnp.concatenate([edge_index[1], jnp.zeros((pad,), jnp.int32)])
    posx = positions[:, 0]
    posy = positions[:, 1]
    posz = positions[:, 2]

    ne = emb_table.shape[0]
    td_dim = emb_table.shape[1]
    hid = W1.shape[0]
    a_d = emb_table @ W1[:, :td_dim].T + b1
    b_s = emb_table @ W1[:, td_dim:2 * td_dim].T
    w18 = jnp.concatenate([W1[:, 2 * td_dim:], a_d.T, b_s.T], axis=1)
    eye8 = jnp.eye(8, dtype=jnp.float32)
    w1k = jnp.kron(w18, eye8)
    w2k = jnp.kron(W2, eye8)
    b2k = jnp.repeat(b2, 8).reshape(8 * hid, 1)
    w3k = jnp.kron(W3, eye8)

    r2p, prp = _sc_edge_gather(posx, posy, posz, type_indices, src, dst, ne)
    out = _tc_energy(r2p, prp, w1k, w2k, b2k, w3k, b3.reshape(1, 1),
                     n_edges, ne)
    return out[0, 0]
